# baseline (speedup 1.0000x reference)
import functools

import jax
import jax.numpy as jnp
from jax import lax
from jax.experimental import pallas as pl
from jax.experimental.pallas import tpu as pltpu
from jax.experimental.pallas import tpu_sc as plsc

VOCAB = 30522
DIM = 768
B = 4096
L = 200

NC = 2
NS = 16
NW = NC * NS
BPW = B // NW
CH = 40
NCHUNK = L // CH
NV = DIM // 16


def _pool_sc(tokens, emb):
    mesh = plsc.VectorSubcoreMesh(core_axis_name="c", subcore_axis_name="s")

    @functools.partial(
        pl.kernel,
        mesh=mesh,
        out_type=jax.ShapeDtypeStruct((B, DIM), jnp.float32),
        scratch_types=[
            pltpu.VMEM((BPW * L,), jnp.int32),
            pltpu.VMEM((CH, DIM), jnp.float32),
            pltpu.VMEM((DIM,), jnp.float32),
            pltpu.SemaphoreType.DMA,
            pltpu.SemaphoreType.DMA,
        ],
    )
    def pool(tokens_hbm, emb_hbm, out_hbm, ids_v, buf_v, acc_v, gsem, osem):
        wid = lax.axis_index("s") * NC + lax.axis_index("c")
        base = pl.multiple_of(wid * BPW, BPW)
        pltpu.sync_copy(tokens_hbm.at[pl.ds(base * L, BPW * L)], ids_v)

        zeros = jnp.zeros((16,), jnp.float32)

        def chunk_body(t, carry):
            i = t // NCHUNK
            c = t % NCHUNK
            idx = ids_v.at[pl.ds(pl.multiple_of(i * L + c * CH, 8), CH)]
            pltpu.async_copy(emb_hbm.at[idx], buf_v, gsem).wait()

            @pl.when(c == 0)
            def _zero():
                for j in range(NV):
                    acc_v[pl.ds(j * 16, 16)] = zeros

            def row_body(r, rc):
                for j in range(NV):
                    plsc.addupdate(acc_v.at[pl.ds(j * 16, 16)],
                                   buf_v[r, pl.ds(j * 16, 16)])
                return rc

            lax.fori_loop(0, CH, row_body, 0)

            @pl.when(c == NCHUNK - 1)
            def _writeback():
                pltpu.async_copy(acc_v, out_hbm.at[base + i], osem).wait()

            return carry

        lax.fori_loop(0, BPW * NCHUNK, chunk_body, 0)

    return pool(tokens, emb)


def _proj_tc(pooled, W, b2d):
    BT = 512

    def body(x_ref, w_ref, b_ref, o_ref):
        x = x_ref[...] * (1.0 / L)
        y = lax.dot_general(x, w_ref[...], (((1,), (1,)), ((), ())),
                            preferred_element_type=jnp.float32)
        y = y + b_ref[...]
        n = jnp.sqrt(jnp.sum(y * y, axis=1, keepdims=True))
        o_ref[...] = y / jnp.maximum(n, 1e-12)

    return pl.pallas_call(
        body,
        grid=(B // BT,),
        in_specs=[
            pl.BlockSpec((BT, DIM), lambda i: (i, 0)),
            pl.BlockSpec((DIM, DIM), lambda i: (0, 0)),
            pl.BlockSpec((1, DIM), lambda i: (0, 0)),
        ],
        out_specs=pl.BlockSpec((BT, DIM), lambda i: (i, 0)),
        out_shape=jax.ShapeDtypeStruct((B, DIM), jnp.float32),
    )(pooled, W, b2d)


def kernel(tokens, emb, W, b):
    tokens = tokens.astype(jnp.int32).reshape(B * L)
    pooled = _pool_sc(tokens, emb)
    return _proj_tc(pooled, W, b.reshape(1, DIM))


# speedup vs baseline: 4.9830x; 4.9830x over previous
import functools

import jax
import jax.numpy as jnp
from jax import lax
from jax.experimental import pallas as pl
from jax.experimental.pallas import tpu as pltpu
from jax.experimental.pallas import tpu_sc as plsc

VOCAB = 30522
DIM = 768
B = 4096
L = 200

NC = 2
NS = 16
NW = NC * NS
BPW = B // NW
CH = 40
NCHUNK = L // CH
NV = DIM // 16


def _pool_sc(tokens, emb):
    mesh = plsc.VectorSubcoreMesh(core_axis_name="c", subcore_axis_name="s")

    @functools.partial(
        pl.kernel,
        mesh=mesh,
        out_type=jax.ShapeDtypeStruct((B, DIM), jnp.float32),
        scratch_types=[
            pltpu.VMEM((BPW * L,), jnp.int32),
            pltpu.VMEM((2, CH, DIM), jnp.float32),
            pltpu.VMEM((DIM,), jnp.float32),
            pltpu.SemaphoreType.DMA,
            pltpu.SemaphoreType.DMA,
            pltpu.SemaphoreType.DMA,
        ],
    )
    def pool(tokens_hbm, emb_hbm, out_hbm, ids_v, buf_v, acc_v, sem0, sem1, osem):
        wid = lax.axis_index("s") * NC + lax.axis_index("c")
        base = pl.multiple_of(wid * BPW, BPW)
        pltpu.sync_copy(tokens_hbm.at[pl.ds(base * L, BPW * L)], ids_v)

        NT = BPW * NCHUNK
        sems = (sem0, sem1)

        def idx_for(t):
            return ids_v.at[pl.ds(pl.multiple_of(t * CH, 8), CH)]

        pltpu.async_copy(emb_hbm.at[idx_for(0)], buf_v.at[0], sem0)

        zero16 = jnp.zeros((16,), jnp.float32)

        def pair_body(p, acc):
            for s in (0, 1):
                t = 2 * p + s

                @pl.when(t + 1 < NT)
                def _issue_next():
                    pltpu.async_copy(emb_hbm.at[idx_for(t + 1)],
                                     buf_v.at[(s + 1) % 2], sems[(s + 1) % 2])

                pltpu.make_async_copy(emb_hbm.at[idx_for(t)],
                                      buf_v.at[s], sems[s]).wait()

                i = t // NCHUNK
                c = t - i * NCHUNK
                acc = tuple(jnp.where(c == 0, zero16, a) for a in acc)

                def row_body(r, a):
                    return tuple(a[j] + buf_v[s, r, pl.ds(j * 16, 16)]
                                 for j in range(NV))

                acc = lax.fori_loop(0, CH, row_body, acc)

                @pl.when(c == NCHUNK - 1)
                def _writeback():
                    for j in range(NV):
                        acc_v[pl.ds(j * 16, 16)] = acc[j]
                    pltpu.async_copy(acc_v, out_hbm.at[base + i], osem).wait()

            return acc

        lax.fori_loop(0, NT // 2, pair_body,
                      tuple(zero16 for _ in range(NV)))

    return pool(tokens, emb)


def _proj_tc(pooled, W, b2d):
    BT = 512

    def body(x_ref, w_ref, b_ref, o_ref):
        x = x_ref[...] * (1.0 / L)
        y = lax.dot_general(x, w_ref[...], (((1,), (1,)), ((), ())),
                            preferred_element_type=jnp.float32)
        y = y + b_ref[...]
        n = jnp.sqrt(jnp.sum(y * y, axis=1, keepdims=True))
        o_ref[...] = y / jnp.maximum(n, 1e-12)

    return pl.pallas_call(
        body,
        grid=(B // BT,),
        in_specs=[
            pl.BlockSpec((BT, DIM), lambda i: (i, 0)),
            pl.BlockSpec((DIM, DIM), lambda i: (0, 0)),
            pl.BlockSpec((1, DIM), lambda i: (0, 0)),
        ],
        out_specs=pl.BlockSpec((BT, DIM), lambda i: (i, 0)),
        out_shape=jax.ShapeDtypeStruct((B, DIM), jnp.float32),
    )(pooled, W, b2d)


def kernel(tokens, emb, W, b):
    tokens = tokens.astype(jnp.int32).reshape(B * L)
    pooled = _pool_sc(tokens, emb)
    return _proj_tc(pooled, W, b.reshape(1, DIM))
